# Initial kernel scaffold; baseline (speedup 1.0000x reference)
import jax
import jax.numpy as jnp
from jax.experimental import pallas as pl


def kernel(prev_input, target, rot_emb):
    raise NotImplementedError("write your pallas kernel here")


# speedup vs baseline: 1.1423x; 1.1423x over previous
import jax
import jax.numpy as jnp
from jax.experimental import pallas as pl

_NUM_CODE = 1024
_CODE_DIM = 64
_TB = 256


def _vq_block(pi_ref, tg_ref, re_ref, ret_ref, out_ref):
    eps = jnp.finfo(jnp.float32).eps
    pib = pi_ref[0]
    tgb = tg_ref[0]
    re = re_ref[...]
    ret = ret_ref[...]

    n2 = jnp.sum(re * re, axis=1, keepdims=True) + eps
    ren = re / jnp.sqrt(n2)
    ci = jax.lax.broadcasted_iota(jnp.int32, (_NUM_CODE, _CODE_DIM), 1)
    ren = jnp.where(ci == 0, 0.0, ren)
    n2t = jnp.sum(ret * ret, axis=0, keepdims=True) + eps
    rent = ret / jnp.sqrt(n2t)
    ri = jax.lax.broadcasted_iota(jnp.int32, (_CODE_DIM, _NUM_CODE), 0)
    rent = jnp.where(ri == 0, 0.0, rent)

    def mm(a, b):
        return jax.lax.dot_general(
            a, b, (((1,), (0,)), ((), ())),
            preferred_element_type=jnp.float32,
            precision=jax.lax.Precision.DEFAULT)

    t1 = mm(ren, tgb)
    p = mm(ren, pib)
    c = 2.0 - 2.0 * jnp.sum(tgb * pib, axis=0, keepdims=True)
    d = c + 4.0 * t1 * p

    dmin = jnp.min(d, axis=0, keepdims=True)
    iota = jax.lax.broadcasted_iota(jnp.int32, (_NUM_CODE, _TB), 0)
    idx = jnp.min(jnp.where(d == dmin, iota, _NUM_CODE),
                  axis=0, keepdims=True)
    s = jnp.where(iota == idx, 2.0 * p, 0.0)

    out_ref[0] = pib - mm(rent, s)


def kernel(prev_input, target, rot_emb):
    B, N, T = prev_input.shape
    grid = (B, T // _TB)
    return pl.pallas_call(
        _vq_block,
        grid=grid,
        in_specs=[
            pl.BlockSpec((1, N, _TB), lambda b, t: (b, 0, t)),
            pl.BlockSpec((1, N, _TB), lambda b, t: (b, 0, t)),
            pl.BlockSpec((_NUM_CODE, _CODE_DIM), lambda b, t: (0, 0)),
            pl.BlockSpec((_CODE_DIM, _NUM_CODE), lambda b, t: (0, 0)),
        ],
        out_specs=pl.BlockSpec((1, N, _TB), lambda b, t: (b, 0, t)),
        out_shape=jax.ShapeDtypeStruct((B, N, T), jnp.float32),
    )(prev_input, target, rot_emb, rot_emb.T)


# speedup vs baseline: 1.2100x; 1.0593x over previous
import jax
import jax.numpy as jnp
from jax.experimental import pallas as pl
from jax.experimental.pallas import tpu as pltpu

_NUM_CODE = 1024
_CODE_DIM = 64
_TB = 256


def _vq_block(pi_ref, tg_ref, re_ref, ret_ref, out_ref, ren_ref, rent2_ref):
    eps = jnp.finfo(jnp.float32).eps

    @pl.when(jnp.logical_and(pl.program_id(0) == 0, pl.program_id(1) == 0))
    def _normalize_codebook():
        re = re_ref[...]
        n2 = jnp.sum(re * re, axis=1, keepdims=True) + eps
        ren = re / jnp.sqrt(n2)
        ci = jax.lax.broadcasted_iota(jnp.int32, (_NUM_CODE, _CODE_DIM), 1)
        ren_ref[...] = jnp.where(ci == 0, 0.0, ren)
        ret = ret_ref[...]
        n2t = jnp.sum(ret * ret, axis=0, keepdims=True) + eps
        rent = ret / jnp.sqrt(n2t)
        ri = jax.lax.broadcasted_iota(jnp.int32, (_CODE_DIM, _NUM_CODE), 0)
        rent2_ref[...] = jnp.where(ri == 0, 0.0, 2.0 * rent)

    pib = pi_ref[0]
    tgb = tg_ref[0]
    ren = ren_ref[...]

    def mm(a, b):
        return jax.lax.dot_general(
            a, b, (((1,), (0,)), ((), ())),
            preferred_element_type=jnp.float32,
            precision=jax.lax.Precision.DEFAULT)

    t1 = mm(ren, tgb)
    p = mm(ren, pib)
    c = 2.0 - 2.0 * jnp.sum(tgb * pib, axis=0, keepdims=True)
    d = c + 4.0 * t1 * p

    dmin = jnp.min(d, axis=0, keepdims=True)
    iota = jax.lax.broadcasted_iota(jnp.int32, (_NUM_CODE, _TB), 0)
    idx = jnp.min(jnp.where(d == dmin, iota, _NUM_CODE),
                  axis=0, keepdims=True)
    s = jnp.where(iota == idx, p, 0.0)

    out_ref[0] = pib - mm(rent2_ref[...], s)


def kernel(prev_input, target, rot_emb):
    B, N, T = prev_input.shape
    grid = (B, T // _TB)
    return pl.pallas_call(
        _vq_block,
        grid=grid,
        in_specs=[
            pl.BlockSpec((1, N, _TB), lambda b, t: (b, 0, t)),
            pl.BlockSpec((1, N, _TB), lambda b, t: (b, 0, t)),
            pl.BlockSpec((_NUM_CODE, _CODE_DIM), lambda b, t: (0, 0)),
            pl.BlockSpec((_CODE_DIM, _NUM_CODE), lambda b, t: (0, 0)),
        ],
        out_specs=pl.BlockSpec((1, N, _TB), lambda b, t: (b, 0, t)),
        out_shape=jax.ShapeDtypeStruct((B, N, T), jnp.float32),
        scratch_shapes=[
            pltpu.VMEM((_NUM_CODE, _CODE_DIM), jnp.float32),
            pltpu.VMEM((_CODE_DIM, _NUM_CODE), jnp.float32),
        ],
    )(prev_input, target, rot_emb, rot_emb.T)


# speedup vs baseline: 1.4991x; 1.2390x over previous
import jax
import jax.numpy as jnp
from jax.experimental import pallas as pl
from jax.experimental.pallas import tpu as pltpu

_NUM_CODE = 1024
_CODE_DIM = 64
_TB = 512


def _vq_block(pi_ref, tg_ref, re_ref, ret_ref, out_ref, ren_ref, rent2_ref):
    eps = jnp.finfo(jnp.float32).eps

    @pl.when(jnp.logical_and(pl.program_id(0) == 0, pl.program_id(1) == 0))
    def _normalize_codebook():
        re = re_ref[...]
        n2 = jnp.sum(re * re, axis=1, keepdims=True) + eps
        ren = re / jnp.sqrt(n2)
        ci = jax.lax.broadcasted_iota(jnp.int32, (_NUM_CODE, _CODE_DIM), 1)
        ren_ref[...] = jnp.where(ci == 0, 0.0, ren)
        ret = ret_ref[...]
        n2t = jnp.sum(ret * ret, axis=0, keepdims=True) + eps
        rent = ret / jnp.sqrt(n2t)
        ri = jax.lax.broadcasted_iota(jnp.int32, (_CODE_DIM, _NUM_CODE), 0)
        rent2_ref[...] = jnp.where(ri == 0, 0.0, 2.0 * rent)

    pib = pi_ref[0]
    tgb = tg_ref[0]
    ren = ren_ref[...]

    def mm(a, b):
        return jax.lax.dot_general(
            a, b, (((1,), (0,)), ((), ())),
            preferred_element_type=jnp.float32,
            precision=jax.lax.Precision.DEFAULT)

    t1 = mm(ren, tgb)
    p = mm(ren, pib)
    c = 2.0 - 2.0 * jnp.sum(tgb * pib, axis=0, keepdims=True)
    d = c + 4.0 * t1 * p

    dmin = jnp.min(d, axis=0, keepdims=True)
    iota = jax.lax.broadcasted_iota(jnp.int32, (_NUM_CODE, _TB), 0)
    idx = jnp.min(jnp.where(d == dmin, iota, _NUM_CODE),
                  axis=0, keepdims=True)
    s = jnp.where(iota == idx, p, 0.0)

    out_ref[0] = pib - mm(rent2_ref[...], s)


def kernel(prev_input, target, rot_emb):
    B, N, T = prev_input.shape
    grid = (B, T // _TB)
    return pl.pallas_call(
        _vq_block,
        grid=grid,
        in_specs=[
            pl.BlockSpec((1, N, _TB), lambda b, t: (b, 0, t)),
            pl.BlockSpec((1, N, _TB), lambda b, t: (b, 0, t)),
            pl.BlockSpec((_NUM_CODE, _CODE_DIM), lambda b, t: (0, 0)),
            pl.BlockSpec((_CODE_DIM, _NUM_CODE), lambda b, t: (0, 0)),
        ],
        out_specs=pl.BlockSpec((1, N, _TB), lambda b, t: (b, 0, t)),
        out_shape=jax.ShapeDtypeStruct((B, N, T), jnp.float32),
        scratch_shapes=[
            pltpu.VMEM((_NUM_CODE, _CODE_DIM), jnp.float32),
            pltpu.VMEM((_CODE_DIM, _NUM_CODE), jnp.float32),
        ],
    )(prev_input, target, rot_emb, rot_emb.T)


# speedup vs baseline: 1.6808x; 1.1212x over previous
import jax
import jax.numpy as jnp
from jax.experimental import pallas as pl
from jax.experimental.pallas import tpu as pltpu

_NUM_CODE = 1024
_CODE_DIM = 64
_TB = 1024


def _vq_block(pi_ref, tg_ref, re_ref, ret_ref, out_ref, ren_ref, rent2_ref):
    eps = jnp.finfo(jnp.float32).eps

    @pl.when(jnp.logical_and(pl.program_id(0) == 0, pl.program_id(1) == 0))
    def _normalize_codebook():
        re = re_ref[...]
        n2 = jnp.sum(re * re, axis=1, keepdims=True) + eps
        ren = re / jnp.sqrt(n2)
        ci = jax.lax.broadcasted_iota(jnp.int32, (_NUM_CODE, _CODE_DIM), 1)
        ren_ref[...] = jnp.where(ci == 0, 0.0, ren)
        ret = ret_ref[...]
        n2t = jnp.sum(ret * ret, axis=0, keepdims=True) + eps
        rent = ret / jnp.sqrt(n2t)
        ri = jax.lax.broadcasted_iota(jnp.int32, (_CODE_DIM, _NUM_CODE), 0)
        rent2_ref[...] = jnp.where(ri == 0, 0.0, 2.0 * rent)

    pib = pi_ref[0]
    tgb = tg_ref[0]
    ren = ren_ref[...]

    def mm(a, b):
        return jax.lax.dot_general(
            a, b, (((1,), (0,)), ((), ())),
            preferred_element_type=jnp.float32,
            precision=jax.lax.Precision.DEFAULT)

    t1 = mm(ren, tgb)
    p = mm(ren, pib)
    c = 2.0 - 2.0 * jnp.sum(tgb * pib, axis=0, keepdims=True)
    d = c + 4.0 * t1 * p

    dmin = jnp.min(d, axis=0, keepdims=True)
    iota = jax.lax.broadcasted_iota(jnp.int32, (_NUM_CODE, _TB), 0)
    idx = jnp.min(jnp.where(d == dmin, iota, _NUM_CODE),
                  axis=0, keepdims=True)
    s = jnp.where(iota == idx, p, 0.0)

    out_ref[0] = pib - mm(rent2_ref[...], s)


def kernel(prev_input, target, rot_emb):
    B, N, T = prev_input.shape
    grid = (B, T // _TB)
    return pl.pallas_call(
        _vq_block,
        grid=grid,
        in_specs=[
            pl.BlockSpec((1, N, _TB), lambda b, t: (b, 0, t)),
            pl.BlockSpec((1, N, _TB), lambda b, t: (b, 0, t)),
            pl.BlockSpec((_NUM_CODE, _CODE_DIM), lambda b, t: (0, 0)),
            pl.BlockSpec((_CODE_DIM, _NUM_CODE), lambda b, t: (0, 0)),
        ],
        out_specs=pl.BlockSpec((1, N, _TB), lambda b, t: (b, 0, t)),
        out_shape=jax.ShapeDtypeStruct((B, N, T), jnp.float32),
        scratch_shapes=[
            pltpu.VMEM((_NUM_CODE, _CODE_DIM), jnp.float32),
            pltpu.VMEM((_CODE_DIM, _NUM_CODE), jnp.float32),
        ],
    )(prev_input, target, rot_emb, rot_emb.T)
